# baseline (speedup 1.0000x reference)
import functools

import numpy as np

import jax
import jax.numpy as jnp
from jax import lax
from jax.experimental import pallas as pl
from jax.experimental.pallas import tpu as pltpu
from jax.experimental.pallas import tpu_sc as plsc

N = 10000
E = 320000
D = 128
NC, NS, L = 2, 16, 16
NW = NC * NS
K = 128
NCHUNK = 79
EPT = NCHUNK * K
NCH0 = 79
NCH1 = 79
TOTCH = NS * (NCH0 + NCH1)
NPAD = 10240
RPT = NPAD // NS
KO = 128

_mesh = plsc.VectorSubcoreMesh(core_axis_name="c", subcore_axis_name="s")


@functools.partial(
    pl.kernel,
    mesh=_mesh,
    out_type=jax.ShapeDtypeStruct((NC, NPAD), jnp.float32),
    scratch_types=[
        pltpu.VMEM((NCHUNK, K), jnp.int32),
        pltpu.VMEM((RPT,), jnp.float32),
        pltpu.VMEM((K,), jnp.float32),
        pltpu.VMEM_SHARED((NPAD,), jnp.float32),
    ],
)
def _deg_kernel(rowp_hbm, out, ridx, zbuf, ones, deg_sp):
    cid = lax.axis_index("c")
    sid = lax.axis_index("s")
    wid = cid * NS + sid
    zero16 = jnp.zeros((16,), jnp.float32)
    one16 = jnp.full((16,), 1.0, jnp.float32)
    for i in range(RPT // 16):
        zbuf[pl.ds(i * 16, 16)] = zero16
    for i in range(K // 16):
        ones[pl.ds(i * 16, 16)] = one16
    pltpu.sync_copy(zbuf, deg_sp.at[pl.ds(sid * RPT, RPT)])
    plsc.subcore_barrier()
    pltpu.sync_copy(rowp_hbm.at[wid], ridx)

    def body(j, carry):
        pltpu.sync_copy(ones, deg_sp.at[ridx.at[j]], add=True)
        return carry

    lax.fori_loop(0, NCHUNK, body, 0)
    plsc.subcore_barrier()
    sl = pl.ds(sid * RPT, RPT)
    pltpu.sync_copy(deg_sp.at[sl], out.at[cid, sl])


@functools.partial(
    pl.kernel,
    mesh=_mesh,
    out_type=jax.ShapeDtypeStruct((NC, NPAD, D), jnp.float32),
    scratch_types=[
        pltpu.VMEM((NCHUNK, K), jnp.int32),
        pltpu.VMEM((1, K), jnp.int32),
        pltpu.VMEM((1, K), jnp.int32),
        pltpu.VMEM((K, D), jnp.float32),
        pltpu.VMEM((K, D), jnp.float32),
        pltpu.VMEM_SHARED((NPAD, D), jnp.float32),
        pltpu.SemaphoreType.DMA,
        pltpu.SemaphoreType.DMA,
        pltpu.SemaphoreType.DMA,
        pltpu.SemaphoreType.DMA,
        pltpu.SemaphoreType.DMA,
        pltpu.SemaphoreType.DMA,
    ],
)
def _spmm_kernel(colp_hbm, rowp_hbm, scaled_hbm, out,
                 ridx, cbuf0, cbuf1, gbuf0, gbuf1, acc,
                 semi0, semi1, semg0, semg1, sems0, sems1):
    cid = lax.axis_index("c")
    sid = lax.axis_index("s")
    wid = cid * NS + sid
    zero16 = jnp.zeros((16,), jnp.float32)

    def zbody(r, carry):
        for q in range(D // 16):
            gbuf0[r, pl.ds(q * 16, 16)] = zero16
        return carry

    lax.fori_loop(0, K, zbody, 0)
    for t in range(RPT // KO):
        pltpu.sync_copy(gbuf0, acc.at[pl.ds(sid * RPT + t * KO, KO)])
    plsc.subcore_barrier()

    pltpu.sync_copy(rowp_hbm.at[wid], ridx)
    pltpu.sync_copy(colp_hbm.at[wid, pl.ds(0, 1)], cbuf0)
    pltpu.async_copy(scaled_hbm.at[cbuf0.at[0]], gbuf0, semg0)
    pltpu.async_copy(colp_hbm.at[wid, pl.ds(1, 1)], cbuf1, semi1)

    pltpu.make_async_copy(colp_hbm.at[wid, pl.ds(1, 1)], cbuf1, semi1).wait()
    pltpu.async_copy(scaled_hbm.at[cbuf1.at[0]], gbuf1, semg1)
    pltpu.make_async_copy(scaled_hbm.at[cbuf0.at[0]], gbuf0, semg0).wait()
    pltpu.async_copy(colp_hbm.at[wid, pl.ds(2, 1)], cbuf0, semi0)
    pltpu.async_copy(gbuf0, acc.at[ridx.at[0]], sems0, add=True)

    def body(i, carry):
        c = 2 * i + 1
        pltpu.make_async_copy(gbuf0, acc.at[ridx.at[c - 1]], sems0).wait()
        pltpu.make_async_copy(colp_hbm.at[wid, pl.ds(c + 1, 1)], cbuf0, semi0).wait()
        pltpu.async_copy(scaled_hbm.at[cbuf0.at[0]], gbuf0, semg0)
        pltpu.make_async_copy(scaled_hbm.at[cbuf1.at[0]], gbuf1, semg1).wait()
        pltpu.async_copy(colp_hbm.at[wid, pl.ds(c + 2, 1)], cbuf1, semi1)
        pltpu.async_copy(gbuf1, acc.at[ridx.at[c]], sems1, add=True)
        pltpu.make_async_copy(gbuf1, acc.at[ridx.at[c]], sems1).wait()
        pltpu.make_async_copy(colp_hbm.at[wid, pl.ds(c + 2, 1)], cbuf1, semi1).wait()
        pltpu.async_copy(scaled_hbm.at[cbuf1.at[0]], gbuf1, semg1)
        pltpu.make_async_copy(scaled_hbm.at[cbuf0.at[0]], gbuf0, semg0).wait()
        pltpu.async_copy(colp_hbm.at[wid, pl.ds(c + 3, 1)], cbuf0, semi0)
        pltpu.async_copy(gbuf0, acc.at[ridx.at[c + 1]], sems0, add=True)
        return carry

    lax.fori_loop(0, (NCHUNK - 1) // 2, body, 0)
    pltpu.make_async_copy(gbuf0, acc.at[ridx.at[0]], sems0).wait()
    pltpu.make_async_copy(scaled_hbm.at[cbuf1.at[0]], gbuf1, semg1).wait()
    pltpu.make_async_copy(colp_hbm.at[wid, pl.ds(0, 1)], cbuf0, semi0).wait()
    plsc.subcore_barrier()
    for t in range(RPT // KO):
        sl = pl.ds(sid * RPT + t * KO, KO)
        pltpu.sync_copy(acc.at[sl], out.at[cid, sl])


_RB = 2000

def _scale_body(feat_ref, d0_ref, d1_ref, scaled_ref, dsq_ref):
    deg = d0_ref[...] + d1_ref[...]
    dsq = lax.rsqrt(jnp.maximum(deg, 1e-12))
    scaled_ref[...] = feat_ref[...] * dsq
    dsq_ref[...] = dsq


_scale_call = pl.pallas_call(
    _scale_body,
    grid=(N // _RB,),
    in_specs=[
        pl.BlockSpec((_RB, D), lambda i: (i, 0)),
        pl.BlockSpec((_RB, 1), lambda i: (i, 0)),
        pl.BlockSpec((_RB, 1), lambda i: (i, 0)),
    ],
    out_specs=[
        pl.BlockSpec((_RB, D), lambda i: (i, 0)),
        pl.BlockSpec((_RB, 1), lambda i: (i, 0)),
    ],
    out_shape=[
        jax.ShapeDtypeStruct((N, D), jnp.float32),
        jax.ShapeDtypeStruct((N, 1), jnp.float32),
    ],
)


def _combine_body(p_ref, dsq_ref, out_ref):
    out_ref[...] = (p_ref[0] + p_ref[1]) * dsq_ref[...]


_combine_call = pl.pallas_call(
    _combine_body,
    grid=(N // _RB,),
    in_specs=[
        pl.BlockSpec((NC, _RB, D), lambda i: (0, i, 0)),
        pl.BlockSpec((_RB, 1), lambda i: (i, 0)),
    ],
    out_specs=pl.BlockSpec((_RB, D), lambda i: (i, 0)),
    out_shape=jax.ShapeDtypeStruct((N, D), jnp.float32),
)


def kernel(features, edge_index, index):
    row = edge_index[0]
    col = edge_index[1]
    pad = NW * EPT - E
    pad_rows = N + (jnp.arange(pad, dtype=jnp.int32) % (NPAD - N))
    rowp = jnp.concatenate([row, pad_rows]).reshape(NW, NCHUNK, K)
    colp = jnp.concatenate(
        [col, jnp.zeros((pad,), jnp.int32)]).reshape(NW, NCHUNK, K)
    colp = jnp.concatenate(
        [colp, jnp.zeros((NW, 2, K), jnp.int32)], axis=1)
    deg = _deg_kernel(rowp)
    d0 = deg[0, :N].reshape(N, 1)
    d1 = deg[1, :N].reshape(N, 1)
    scaled, dsq = _scale_call(features, d0, d1)
    p = _spmm_kernel(colp, rowp, scaled)
    return _combine_call(p, dsq)


# speedup vs baseline: 1.4182x; 1.4182x over previous
import functools

import numpy as np

import jax
import jax.numpy as jnp
from jax import lax
from jax.experimental import pallas as pl
from jax.experimental.pallas import tpu as pltpu
from jax.experimental.pallas import tpu_sc as plsc

N = 10000
E = 320000
D = 128
NC, NS, L = 2, 16, 16
NW = NC * NS
K = 128
NCHUNK = 79
EPT = NCHUNK * K
NPAD = 10240
RPT = NPAD // NS
KO = 128

_mesh = plsc.VectorSubcoreMesh(core_axis_name="c", subcore_axis_name="s")


@functools.partial(
    pl.kernel,
    mesh=_mesh,
    out_type=jax.ShapeDtypeStruct((NC, NPAD), jnp.float32),
    scratch_types=[
        pltpu.VMEM((NCHUNK, K), jnp.int32),
        pltpu.VMEM((RPT,), jnp.float32),
        pltpu.VMEM((K,), jnp.float32),
        pltpu.VMEM_SHARED((NPAD,), jnp.float32),
    ],
)
def _deg_kernel(rowp_hbm, out, ridx, zbuf, ones, deg_sp):
    cid = lax.axis_index("c")
    sid = lax.axis_index("s")
    wid = cid * NS + sid
    zero16 = jnp.zeros((16,), jnp.float32)
    one16 = jnp.full((16,), 1.0, jnp.float32)
    for i in range(RPT // 16):
        zbuf[pl.ds(i * 16, 16)] = zero16
    for i in range(K // 16):
        ones[pl.ds(i * 16, 16)] = one16
    pltpu.sync_copy(zbuf, deg_sp.at[pl.ds(sid * RPT, RPT)])
    plsc.subcore_barrier()
    pltpu.sync_copy(rowp_hbm.at[wid], ridx)

    def body(j, carry):
        pltpu.sync_copy(ones, deg_sp.at[ridx.at[j]], add=True)
        return carry

    lax.fori_loop(0, NCHUNK, body, 0)
    plsc.subcore_barrier()
    sl = pl.ds(sid * RPT, RPT)
    pltpu.sync_copy(deg_sp.at[sl], out.at[cid, sl])


@functools.partial(
    pl.kernel,
    mesh=_mesh,
    out_type=jax.ShapeDtypeStruct((NC, NPAD, D), jnp.float32),
    scratch_types=[
        pltpu.VMEM((2, K), jnp.int32),
        pltpu.VMEM((2, K), jnp.int32),
        pltpu.VMEM((K, D), jnp.float32),
        pltpu.VMEM((K, D), jnp.float32),
        pltpu.VMEM_SHARED((NPAD, D), jnp.float32),
        pltpu.SemaphoreType.DMA,
        pltpu.SemaphoreType.DMA,
        pltpu.SemaphoreType.DMA,
        pltpu.SemaphoreType.DMA,
    ],
)
def _spmm_kernel(eidx_hbm, scaled_hbm, out,
                 ibuf0, ibuf1, gbuf0, gbuf1, acc, semi0, semi1, semg0, semg1):
    cid = lax.axis_index("c")
    sid = lax.axis_index("s")
    wid = cid * NS + sid
    zero16 = jnp.zeros((16,), jnp.float32)

    def zbody(r, carry):
        for q in range(D // 16):
            gbuf0[r, pl.ds(q * 16, 16)] = zero16
        return carry

    lax.fori_loop(0, K, zbody, 0)
    for t in range(RPT // KO):
        pltpu.sync_copy(gbuf0, acc.at[pl.ds(sid * RPT + t * KO, KO)])
    plsc.subcore_barrier()

    pltpu.sync_copy(eidx_hbm.at[wid, 0], ibuf0)
    pltpu.async_copy(scaled_hbm.at[ibuf0.at[0]], gbuf0, semg0)
    pltpu.async_copy(eidx_hbm.at[wid, 1], ibuf1, semi1)

    def body(i, carry):
        j0 = 2 * i
        pltpu.make_async_copy(eidx_hbm.at[wid, j0 + 1], ibuf1, semi1).wait()
        pltpu.make_async_copy(scaled_hbm.at[ibuf0.at[0]], gbuf0, semg0).wait()
        pltpu.async_copy(scaled_hbm.at[ibuf1.at[0]], gbuf1, semg1)
        pltpu.sync_copy(gbuf0, acc.at[ibuf0.at[1]], add=True)
        pltpu.async_copy(eidx_hbm.at[wid, j0 + 2], ibuf0, semi0)
        pltpu.make_async_copy(eidx_hbm.at[wid, j0 + 2], ibuf0, semi0).wait()
        pltpu.make_async_copy(scaled_hbm.at[ibuf1.at[0]], gbuf1, semg1).wait()
        pltpu.async_copy(scaled_hbm.at[ibuf0.at[0]], gbuf0, semg0)
        pltpu.sync_copy(gbuf1, acc.at[ibuf1.at[1]], add=True)
        pltpu.async_copy(eidx_hbm.at[wid, j0 + 3], ibuf1, semi1)
        return carry

    lax.fori_loop(0, (NCHUNK - 1) // 2, body, 0)
    pltpu.make_async_copy(eidx_hbm.at[wid, 0], ibuf1, semi1).wait()
    pltpu.make_async_copy(scaled_hbm.at[ibuf0.at[0]], gbuf0, semg0).wait()
    pltpu.sync_copy(gbuf0, acc.at[ibuf0.at[1]], add=True)
    plsc.subcore_barrier()
    for t in range(RPT // KO):
        sl = pl.ds(sid * RPT + t * KO, KO)
        pltpu.sync_copy(acc.at[sl], out.at[cid, sl])


_RB = 2000

def _scale_body(feat_ref, d0_ref, d1_ref, scaled_ref, dsq_ref):
    deg = d0_ref[...] + d1_ref[...]
    dsq = lax.rsqrt(jnp.maximum(deg, 1e-12))
    scaled_ref[...] = feat_ref[...] * dsq
    dsq_ref[...] = dsq


_scale_call = pl.pallas_call(
    _scale_body,
    grid=(N // _RB,),
    in_specs=[
        pl.BlockSpec((_RB, D), lambda i: (i, 0)),
        pl.BlockSpec((_RB, 1), lambda i: (i, 0)),
        pl.BlockSpec((_RB, 1), lambda i: (i, 0)),
    ],
    out_specs=[
        pl.BlockSpec((_RB, D), lambda i: (i, 0)),
        pl.BlockSpec((_RB, 1), lambda i: (i, 0)),
    ],
    out_shape=[
        jax.ShapeDtypeStruct((N, D), jnp.float32),
        jax.ShapeDtypeStruct((N, 1), jnp.float32),
    ],
)


def _combine_body(p_ref, dsq_ref, out_ref):
    out_ref[...] = (p_ref[0] + p_ref[1]) * dsq_ref[...]


_combine_call = pl.pallas_call(
    _combine_body,
    grid=(N // _RB,),
    in_specs=[
        pl.BlockSpec((NC, _RB, D), lambda i: (0, i, 0)),
        pl.BlockSpec((_RB, 1), lambda i: (i, 0)),
    ],
    out_specs=pl.BlockSpec((_RB, D), lambda i: (i, 0)),
    out_shape=jax.ShapeDtypeStruct((N, D), jnp.float32),
)


def kernel(features, edge_index, index):
    row = edge_index[0]
    col = edge_index[1]
    pad = NW * EPT - E
    pad_rows = N + (jnp.arange(pad, dtype=jnp.int32) % (NPAD - N))
    rowp = jnp.concatenate([row, pad_rows]).reshape(NW, NCHUNK, K)
    colp = jnp.concatenate(
        [col, jnp.zeros((pad,), jnp.int32)]).reshape(NW, NCHUNK, K)
    eidx = jnp.concatenate(
        [jnp.stack([colp, rowp], axis=2),
         jnp.zeros((NW, 1, 2, K), jnp.int32)], axis=1)
    deg = _deg_kernel(rowp)
    d0 = deg[0, :N].reshape(N, 1)
    d1 = deg[1, :N].reshape(N, 1)
    scaled, dsq = _scale_call(features, d0, d1)
    p = _spmm_kernel(eidx, scaled)
    return _combine_call(p, dsq)
